# baseline (speedup 1.0000x reference)
import functools

import jax
import jax.numpy as jnp
from jax import lax
from jax.experimental import pallas as pl
from jax.experimental.pallas import tpu as pltpu
from jax.experimental.pallas import tpu_sc as plsc

NC = 2
NS = 16
NW = NC * NS

B_ROWS = 1600
L = 128
D = 128
ROWS_PER_W = B_ROWS // NW


NBUF = 5


def _gather_body(table_hbm, idx_hbm, out_hbm, idx_v, buf,
                 sg0, sg1, sg2, sg3, sg4, ss0, ss1, ss2, ss3, ss4):
    sg = (sg0, sg1, sg2, sg3, sg4)
    ss = (ss0, ss1, ss2, ss3, ss4)
    wid = lax.axis_index("s") * NC + lax.axis_index("c")
    base = wid * ROWS_PER_W
    pltpu.sync_copy(idx_hbm.at[wid], idx_v)

    def start_gather(j, b):
        pltpu.async_copy(table_hbm.at[idx_v.at[j]], buf.at[b], sg[b])

    def wait_gather(j, b):
        pltpu.make_async_copy(
            table_hbm.at[idx_v.at[j]], buf.at[b], sg[b]).wait()

    def start_scatter(j, b):
        pltpu.async_copy(buf.at[b], out_hbm.at[pl.ds((base + j) * L, L)],
                         ss[b])

    def wait_scatter(j, b):
        pltpu.make_async_copy(buf.at[b],
                              out_hbm.at[pl.ds((base + j) * L, L)],
                              ss[b]).wait()

    for j in range(NBUF - 1):
        start_gather(j, j)

    for b in range(NBUF):
        bp = (b - 1) % NBUF
        if b > 0:
            wait_scatter(b - 1, bp)
        start_gather(b + NBUF - 1, bp)
        wait_gather(b, b)
        start_scatter(b, b)

    def outer(i, carry):
        j0 = i * NBUF
        for b in range(NBUF):
            j = j0 + b
            bp = (b - 1) % NBUF
            wait_scatter(j - 1, bp)
            start_gather(j + NBUF - 1, bp)
            wait_gather(j, b)
            start_scatter(j, b)
        return carry

    lax.fori_loop(1, ROWS_PER_W // NBUF - 1, outer, 0)

    for b in range(NBUF):
        j = ROWS_PER_W - NBUF + b
        bp = (b - 1) % NBUF
        wait_scatter(j - 1, bp)
        if b == 0:
            start_gather(j + NBUF - 1, bp)
        wait_gather(j, b)
        start_scatter(j, b)

    wait_scatter(ROWS_PER_W - 1, (ROWS_PER_W - 1) % NBUF)


@jax.jit
def _lookup(table, idx):
    mesh = plsc.VectorSubcoreMesh(core_axis_name="c", subcore_axis_name="s")
    call = functools.partial(
        pl.kernel,
        mesh=mesh,
        out_type=jax.ShapeDtypeStruct((B_ROWS * L, D), jnp.float32),
        scratch_types=[
            pltpu.VMEM((ROWS_PER_W, L), jnp.int32),
            pltpu.VMEM((NBUF, L, D), jnp.float32),
        ] + [pltpu.SemaphoreType.DMA] * (2 * NBUF),
    )(_gather_body)
    return call(table, idx)


def kernel(time_features, table):
    table = table.at[0].set(0.0)
    idx = time_features.reshape(NW, ROWS_PER_W, L)
    out = _lookup(table, idx)
    return out.reshape(4096, 50, D)


# speedup vs baseline: 1.5940x; 1.5940x over previous
import functools

import jax
import jax.numpy as jnp
from jax import lax
from jax.experimental import pallas as pl
from jax.experimental.pallas import tpu as pltpu
from jax.experimental.pallas import tpu_sc as plsc

NC = 2
NS = 16
NW = NC * NS

B = 4096
T = 50
D = 128
ROWS_PER_W = B // NW
BK = 8
NSTEPS = ROWS_PER_W // BK


def _gather_body(table_hbm, idx_hbm, out_hbm, idx_v, buf, sg):
    wid = lax.axis_index("s") * NC + lax.axis_index("c")
    base = wid * ROWS_PER_W
    pltpu.sync_copy(idx_hbm.at[pl.ds(base, ROWS_PER_W)], idx_v)

    def start_gathers(s, p):
        for i in range(BK):
            pltpu.async_copy(
                table_hbm.at[idx_v.at[s * BK + i]], buf.at[p, i], sg)

    def drain_gathers(s, p):
        for i in range(BK):
            pltpu.make_async_copy(
                table_hbm.at[idx_v.at[s * BK + i]], buf.at[p, i], sg).wait()

    def scatter(s, p):
        pltpu.sync_copy(buf.at[p], out_hbm.at[pl.ds(base + s * BK, BK)])

    start_gathers(0, 0)

    def step(s, carry):
        p = lax.rem(s, 2)
        drain_gathers(s, p)
        start_gathers(s + 1, 1 - p)
        scatter(s, p)
        return carry

    lax.fori_loop(0, NSTEPS - 1, step, 0)

    p_last = (NSTEPS - 1) % 2
    drain_gathers(NSTEPS - 1, p_last)
    scatter(NSTEPS - 1, p_last)


@jax.jit
def _lookup(table, idx):
    mesh = plsc.VectorSubcoreMesh(core_axis_name="c", subcore_axis_name="s")
    call = functools.partial(
        pl.kernel,
        mesh=mesh,
        out_type=jax.ShapeDtypeStruct((B, T, D), jnp.float32),
        scratch_types=[
            pltpu.VMEM((ROWS_PER_W, T), jnp.int32),
            pltpu.VMEM((2, BK, T, D), jnp.float32),
            pltpu.SemaphoreType.DMA,
        ],
    )(_gather_body)
    return call(table, idx)


def kernel(time_features, table):
    table = table.at[0].set(0.0)
    return _lookup(table, time_features)


# speedup vs baseline: 2.3536x; 1.4766x over previous
import functools

import jax
import jax.numpy as jnp
from jax import lax
from jax.experimental import pallas as pl
from jax.experimental.pallas import tpu as pltpu
from jax.experimental.pallas import tpu_sc as plsc

NC = 2
NS = 16
NW = NC * NS

B = 4096
T = 50
D = 128
W = B // NW
NBUF = 5


def _gather_body(table_hbm, idx_hbm, out_hbm, idx_v, buf,
                 sg0, sg1, sg2, sg3, sg4):
    sg = (sg0, sg1, sg2, sg3, sg4)
    wid = lax.axis_index("s") * NC + lax.axis_index("c")
    base = wid * W
    pltpu.sync_copy(idx_hbm.at[:, pl.ds(base, W)], idx_v)

    def start_gather(t, b):
        pltpu.async_copy(table_hbm.at[idx_v.at[t]], buf.at[b], sg[b])

    def wait_gather(t, b):
        pltpu.make_async_copy(
            table_hbm.at[idx_v.at[t]], buf.at[b], sg[b]).wait()

    def scatter(t, b):
        pltpu.sync_copy(buf.at[b], out_hbm.at[t, pl.ds(base, W)])

    for b in range(NBUF):
        start_gather(b, b)

    def outer(i, carry):
        t0 = i * NBUF
        for b in range(NBUF):
            t = t0 + b
            wait_gather(t, b)
            scatter(t, b)
            start_gather(t + NBUF, b)
        return carry

    lax.fori_loop(0, T // NBUF - 1, outer, 0)

    for b in range(NBUF):
        t = T - NBUF + b
        wait_gather(t, b)
        scatter(t, b)


@jax.jit
def _lookup(time_features, table):
    table = table.at[0].set(0.0)
    mesh = plsc.VectorSubcoreMesh(core_axis_name="c", subcore_axis_name="s")
    call = functools.partial(
        pl.kernel,
        mesh=mesh,
        out_type=jax.ShapeDtypeStruct((T, B, D), jnp.float32),
        scratch_types=[
            pltpu.VMEM((T, W), jnp.int32),
            pltpu.VMEM((NBUF, W, D), jnp.float32),
        ] + [pltpu.SemaphoreType.DMA] * NBUF,
    )(_gather_body)
    out_tbd = call(table, time_features.T)
    return jnp.transpose(out_tbd, (1, 0, 2))


def kernel(time_features, table):
    return _lookup(time_features, table)


# speedup vs baseline: 5.4858x; 2.3308x over previous
import functools

import jax
import jax.numpy as jnp
from jax import lax
from jax.experimental import pallas as pl
from jax.experimental.pallas import tpu as pltpu
from jax.experimental.pallas import tpu_sc as plsc

NC = 2
NS = 16
NW = NC * NS

B = 4096
T = 50
D = 128
W = B // NW
NBUF = 5


VP = 1024
ROWS_PER_TILE = VP // NS


def _gather_body(table_hbm, idx_hbm, out_hbm, idx_v, buf, tab_sh,
                 sg0, sg1, sg2, sg3, sg4):
    sg = (sg0, sg1, sg2, sg3, sg4)
    sid = lax.axis_index("s")
    wid = sid * NC + lax.axis_index("c")
    base = wid * W

    pltpu.sync_copy(table_hbm.at[pl.ds(sid * ROWS_PER_TILE, ROWS_PER_TILE)],
                    tab_sh.at[pl.ds(sid * ROWS_PER_TILE, ROWS_PER_TILE)])
    pltpu.sync_copy(idx_hbm.at[:, pl.ds(base, W)], idx_v)
    plsc.subcore_barrier()

    def start_gather(t, b):
        pltpu.async_copy(tab_sh.at[idx_v.at[t]], buf.at[b], sg[b])

    def wait_gather(t, b):
        pltpu.make_async_copy(
            tab_sh.at[idx_v.at[t]], buf.at[b], sg[b]).wait()

    def scatter(t, b):
        pltpu.sync_copy(buf.at[b], out_hbm.at[t, pl.ds(base, W)])

    for b in range(NBUF):
        start_gather(b, b)

    def outer(i, carry):
        t0 = i * NBUF
        for b in range(NBUF):
            t = t0 + b
            wait_gather(t, b)
            scatter(t, b)
            start_gather(t + NBUF, b)
        return carry

    lax.fori_loop(0, T // NBUF - 1, outer, 0)

    for b in range(NBUF):
        t = T - NBUF + b
        wait_gather(t, b)
        scatter(t, b)


@jax.jit
def _lookup(time_features, table):
    table = table.at[0].set(0.0)
    mesh = plsc.VectorSubcoreMesh(core_axis_name="c", subcore_axis_name="s")
    call = functools.partial(
        pl.kernel,
        mesh=mesh,
        out_type=jax.ShapeDtypeStruct((T, B, D), jnp.float32),
        scratch_types=[
            pltpu.VMEM((T, W), jnp.int32),
            pltpu.VMEM((NBUF, W, D), jnp.float32),
            pltpu.VMEM_SHARED((VP, D), jnp.float32),
        ] + [pltpu.SemaphoreType.DMA] * NBUF,
    )(_gather_body)
    table_p = jnp.pad(table, ((0, VP - table.shape[0]), (0, 0)))
    out_tbd = call(table_p, time_features.T)
    return jnp.transpose(out_tbd, (1, 0, 2))


def kernel(time_features, table):
    return _lookup(time_features, table)
